# Initial kernel scaffold; baseline (speedup 1.0000x reference)
import functools

import jax
import jax.numpy as jnp
from jax import lax
from jax.experimental import pallas as pl
from jax.experimental.pallas import tpu as pltpu
from jax.experimental.pallas import tpu_sc as plsc

N_DST = 10000
D = 128
CHUNK = 125
NC, NS = 2, 16
NW = NC * NS
ROWS_PER_TILE = N_DST // NS


def _sc_partials(src_emb, idx2d, n_edges):
    E = n_edges
    epw = E // NW
    cpw = epw // CHUNK
    assert epw * NW == E and cpw * CHUNK == epw and cpw % 2 == 0
    mesh = plsc.VectorSubcoreMesh(core_axis_name="c", subcore_axis_name="s")

    @functools.partial(
        pl.kernel,
        out_type=jax.ShapeDtypeStruct((NC, N_DST, D), jnp.float32),
        mesh=mesh,
        scratch_types=[
            pltpu.VMEM_SHARED((N_DST, D), jnp.float32),
            pltpu.VMEM((cpw, CHUNK), jnp.int32),
            pltpu.VMEM((2, CHUNK, D), jnp.float32),
            pltpu.VMEM((CHUNK, D), jnp.float32),
            pltpu.SemaphoreType.DMA,
            pltpu.SemaphoreType.DMA,
        ],
    )
    def k(src_hbm, idx_hbm, out_hbm, acc, idx_v, rows_v, zero_v, sem0, sem1):
        c = lax.axis_index("c")
        s = lax.axis_index("s")
        wid = s * NC + c
        r0 = s * ROWS_PER_TILE

        @pl.when(c == 0)
        def _():
            for kk in range(ROWS_PER_TILE // CHUNK):
                pltpu.sync_copy(src_hbm.at[pl.ds(E + r0 + kk * CHUNK, CHUNK)],
                                acc.at[pl.ds(r0 + kk * CHUNK, CHUNK)])

        @pl.when(c == 1)
        def _():
            def zrow(r, carry):
                for col in range(D // 16):
                    zero_v[r, pl.ds(col * 16, 16)] = jnp.zeros((16,), jnp.float32)
                return carry
            lax.fori_loop(0, CHUNK, zrow, 0)
            for kk in range(ROWS_PER_TILE // CHUNK):
                pltpu.sync_copy(zero_v, acc.at[pl.ds(r0 + kk * CHUNK, CHUNK)])

        plsc.subcore_barrier()

        pltpu.sync_copy(idx_hbm.at[pl.ds(wid * cpw, cpw)], idx_v)
        ebase = wid * epw
        sems = (sem0, sem1)

        def gstart(j, b):
            pltpu.async_copy(src_hbm.at[pl.ds(ebase + j * CHUNK, CHUNK)],
                             rows_v.at[b], sems[b])

        def gwait(j, b):
            pltpu.make_async_copy(src_hbm.at[pl.ds(ebase + j * CHUNK, CHUNK)],
                                  rows_v.at[b], sems[b]).wait()

        gstart(0, 0)
        gstart(1, 1)

        def body(g, carry):
            for b in range(2):
                j = g * 2 + b
                gwait(j, b)
                gstart(j + 2, b)
                pltpu.sync_copy(rows_v.at[b], acc.at[idx_v.at[j]], add=True)
            return carry
        lax.fori_loop(0, cpw // 2 - 1, body, 0)
        for b in range(2):
            j = cpw - 2 + b
            gwait(j, b)
            pltpu.sync_copy(rows_v.at[b], acc.at[idx_v.at[j]], add=True)

        plsc.subcore_barrier()
        pltpu.sync_copy(acc.at[pl.ds(r0, ROWS_PER_TILE)],
                        out_hbm.at[c, pl.ds(r0, ROWS_PER_TILE)])

    return k(src_emb, idx2d)


def _combine(partials):
    blk = 1000

    def add_k(p_ref, o_ref):
        o_ref[...] = p_ref[0] + p_ref[1]

    return pl.pallas_call(
        add_k,
        grid=(N_DST // blk,),
        in_specs=[pl.BlockSpec((NC, blk, D), lambda i: (0, i, 0))],
        out_specs=pl.BlockSpec((blk, D), lambda i: (i, 0)),
        out_shape=jax.ShapeDtypeStruct((N_DST, D), jnp.float32),
    )(partials)


def kernel(src_emb, src_emb_in, dst_ids):
    del src_emb_in
    E = dst_ids.shape[0]
    idx2d = dst_ids.astype(jnp.int32).reshape(E // CHUNK, CHUNK)
    partials = _sc_partials(src_emb, idx2d, E)
    return _combine(partials)


# speedup vs baseline: 9.2829x; 9.2829x over previous
import functools

import jax
import jax.numpy as jnp
from jax import lax
from jax.experimental import pallas as pl
from jax.experimental.pallas import tpu as pltpu
from jax.experimental.pallas import tpu_sc as plsc

N_DST = 10000
D = 128
CHUNK = 80
NC, NS = 2, 16
NW = NC * NS
RPT = 624
ZBLK = 16


def _sc_partials(src_emb, idx3d, n_edges):
    E = n_edges
    epw = E // NW
    cpw = epw // CHUNK
    assert epw * NW == E and cpw * CHUNK == epw
    mesh = plsc.VectorSubcoreMesh(core_axis_name="c", subcore_axis_name="s")

    @functools.partial(
        pl.kernel,
        out_type=jax.ShapeDtypeStruct((NC, N_DST, D), jnp.float32),
        mesh=mesh,
        scratch_types=[
            pltpu.VMEM_SHARED((N_DST, D), jnp.float32),
            pltpu.VMEM((cpw, CHUNK), jnp.int32),
            pltpu.VMEM((2, CHUNK, D), jnp.float32),
            pltpu.VMEM((ZBLK, D), jnp.float32),
            pltpu.SemaphoreType.DMA,
            pltpu.SemaphoreType.DMA,
        ],
    )
    def k(src_hbm, idx_hbm, out_hbm, acc, idx_v, rows_v, zero_v, sem0, sem1):
        c = lax.axis_index("c")
        s = lax.axis_index("s")
        wid = s * NC + c
        r0 = s * RPT

        @pl.when(c == 0)
        def _():
            pltpu.sync_copy(src_hbm.at[pl.ds(E + r0, RPT)], acc.at[pl.ds(r0, RPT)])

            @pl.when(s == NS - 1)
            def _():
                pltpu.sync_copy(src_hbm.at[pl.ds(E + NS * RPT, N_DST - NS * RPT)],
                                acc.at[pl.ds(NS * RPT, N_DST - NS * RPT)])

        @pl.when(c == 1)
        def _():
            def zrow(r, carry):
                for col in range(D // 16):
                    zero_v[r, pl.ds(col * 16, 16)] = jnp.zeros((16,), jnp.float32)
                return carry
            lax.fori_loop(0, ZBLK, zrow, 0)
            for kk in range(RPT // ZBLK):
                pltpu.sync_copy(zero_v, acc.at[pl.ds(r0 + kk * ZBLK, ZBLK)])

            @pl.when(s == NS - 1)
            def _():
                pltpu.sync_copy(zero_v.at[pl.ds(0, N_DST - NS * RPT)],
                                acc.at[pl.ds(NS * RPT, N_DST - NS * RPT)])

        plsc.subcore_barrier()

        pltpu.sync_copy(idx_hbm.at[wid], idx_v)
        ebase = wid * epw
        sems = (sem0, sem1)

        def gstart(j, b):
            pltpu.async_copy(src_hbm.at[pl.ds(ebase + j * CHUNK, CHUNK)],
                             rows_v.at[b], sems[b])

        def gwait(j, b):
            pltpu.make_async_copy(src_hbm.at[pl.ds(ebase + j * CHUNK, CHUNK)],
                                  rows_v.at[b], sems[b]).wait()

        def scat(j, b):
            pltpu.sync_copy(rows_v.at[b], acc.at[idx_v.at[j]], add=True)

        gstart(0, 0)
        gstart(1, 1)
        npairs = (cpw - 2) // 2

        def body(g, carry):
            for b in range(2):
                j = g * 2 + b
                gwait(j, b)
                gstart(j + 2, b)
                scat(j, b)
            return carry
        lax.fori_loop(0, npairs, body, 0)
        for j in range(2 * npairs, cpw):
            b = j % 2
            gwait(j, b)
            if j + 2 < cpw:
                gstart(j + 2, b)
            scat(j, b)

        plsc.subcore_barrier()
        pltpu.sync_copy(acc.at[pl.ds(r0, RPT)], out_hbm.at[c, pl.ds(r0, RPT)])

        @pl.when(s == NS - 1)
        def _():
            pltpu.sync_copy(acc.at[pl.ds(NS * RPT, N_DST - NS * RPT)],
                            out_hbm.at[c, pl.ds(NS * RPT, N_DST - NS * RPT)])

    return k(src_emb, idx3d)


def _combine(partials):
    blk = 1000

    def add_k(p_ref, o_ref):
        o_ref[...] = p_ref[0] + p_ref[1]

    return pl.pallas_call(
        add_k,
        grid=(N_DST // blk,),
        in_specs=[pl.BlockSpec((NC, blk, D), lambda i: (0, i, 0))],
        out_specs=pl.BlockSpec((blk, D), lambda i: (i, 0)),
        out_shape=jax.ShapeDtypeStruct((N_DST, D), jnp.float32),
    )(partials)


def kernel(src_emb, src_emb_in, dst_ids):
    del src_emb_in
    E = dst_ids.shape[0]
    epw = E // NW
    idx3d = dst_ids.astype(jnp.int32).reshape(NW, epw // CHUNK, CHUNK)
    partials = _sc_partials(src_emb, idx3d, E)
    return _combine(partials)
